# Initial kernel scaffold; baseline (speedup 1.0000x reference)
import jax
import jax.numpy as jnp
from jax.experimental import pallas as pl


def kernel(x, W, b, gn_weight, gn_bias, gn_mean_scale, edge_index, batch):
    raise NotImplementedError("write your pallas kernel here")


# speedup vs baseline: 20.9085x; 20.9085x over previous
import functools

import jax
import jax.numpy as jnp
from jax import lax
from jax.experimental import pallas as pl
from jax.experimental.pallas import tpu as pltpu
from jax.experimental.pallas import tpu_sc as plsc

N = 10000
NP = 10240
E = 320000
D = 128
G = 64

NC = 2
NS = 16
NW = NC * NS
K = 128
NCHUNK = E // K
FULL_ROUNDS = NCHUNK // NW
TAIL = NCHUNK - FULL_ROUNDS * NW
RPT = NP // NS
DEGW = 8

_mesh = plsc.VectorSubcoreMesh(core_axis_name="c", subcore_axis_name="s")


@functools.partial(
    pl.kernel,
    out_type=jax.ShapeDtypeStruct((NC, NP, DEGW), jnp.float32),
    mesh=_mesh,
    scratch_types=[
        pltpu.VMEM((K,), jnp.int32),
        pltpu.VMEM((K, DEGW), jnp.float32),
        pltpu.VMEM_SHARED((NP, DEGW), jnp.float32),
    ],
)
def _sc_degree(dst_hbm, ones_hbm, zeros_hbm, out_hbm, idx_v, ones_v, acc_sh):
    c = lax.axis_index("c")
    s = lax.axis_index("s")
    w = s * NC + c
    pltpu.sync_copy(zeros_hbm.at[pl.ds(s * RPT, RPT)],
                    acc_sh.at[pl.ds(s * RPT, RPT)])
    pltpu.sync_copy(ones_hbm, ones_v)
    plsc.subcore_barrier()

    def process(cid):
        pltpu.sync_copy(dst_hbm.at[pl.ds(cid * K, K)], idx_v)
        pltpu.sync_copy(ones_v, acc_sh.at[idx_v], add=True)

    def body(j, carry):
        process(w + NW * j)
        return carry

    lax.fori_loop(0, FULL_ROUNDS, body, 0)

    @pl.when(w < TAIL)
    def _():
        process(w + NW * FULL_ROUNDS)

    plsc.subcore_barrier()
    pltpu.sync_copy(acc_sh.at[pl.ds(s * RPT, RPT)],
                    out_hbm.at[c, pl.ds(s * RPT, RPT)])


def _mm_body(x_ref, w_ref, deg_ref, h_ref):
    deg = deg_ref[0, :, 0] + deg_ref[1, :, 0] + 1.0
    dinv = lax.rsqrt(deg)[:, None]
    h_ref[...] = jnp.dot(x_ref[...], w_ref[...],
                         preferred_element_type=jnp.float32) * dinv


def _tc_matmul(x, W, deg2):
    BN = 2048
    return pl.pallas_call(
        _mm_body,
        grid=(NP // BN,),
        in_specs=[
            pl.BlockSpec((BN, D), lambda i: (i, 0)),
            pl.BlockSpec((D, D), lambda i: (0, 0)),
            pl.BlockSpec((NC, BN, DEGW), lambda i: (0, i, 0)),
        ],
        out_specs=pl.BlockSpec((BN, D), lambda i: (i, 0)),
        out_shape=jax.ShapeDtypeStruct((NP, D), jnp.float32),
    )(x, W, deg2)


@functools.partial(
    pl.kernel,
    out_type=jax.ShapeDtypeStruct((NC, NP, D), jnp.float32),
    mesh=_mesh,
    scratch_types=[
        pltpu.VMEM((K,), jnp.int32),
        pltpu.VMEM((K,), jnp.int32),
        pltpu.VMEM((K, D), jnp.float32),
        pltpu.VMEM_SHARED((NP, D), jnp.float32),
        pltpu.SemaphoreType.DMA,
    ],
)
def _sc_scatter(hp_hbm, src_hbm, dst_hbm, zeros_hbm, out_hbm,
                sidx, didx, rows, acc_sh, sem):
    c = lax.axis_index("c")
    s = lax.axis_index("s")
    w = s * NC + c

    @pl.when(c == 0)
    def _():
        pltpu.sync_copy(hp_hbm.at[pl.ds(s * RPT, RPT)],
                        acc_sh.at[pl.ds(s * RPT, RPT)])

    @pl.when(c == 1)
    def _():
        pltpu.sync_copy(zeros_hbm.at[pl.ds(s * RPT, RPT)],
                        acc_sh.at[pl.ds(s * RPT, RPT)])

    plsc.subcore_barrier()

    def process(cid):
        e0 = cid * K
        pltpu.sync_copy(src_hbm.at[pl.ds(e0, K)], sidx)
        pltpu.sync_copy(dst_hbm.at[pl.ds(e0, K)], didx)
        pltpu.async_copy(hp_hbm.at[sidx], rows, sem).wait()
        pltpu.sync_copy(rows, acc_sh.at[didx], add=True)

    def body(j, carry):
        process(w + NW * j)
        return carry

    lax.fori_loop(0, FULL_ROUNDS, body, 0)

    @pl.when(w < TAIL)
    def _():
        process(w + NW * FULL_ROUNDS)

    plsc.subcore_barrier()
    pltpu.sync_copy(acc_sh.at[pl.ds(s * RPT, RPT)],
                    out_hbm.at[c, pl.ds(s * RPT, RPT)])


def _stats_body(acc_ref, deg_ref, b_ref, batch_ref,
                out_ref, segs_ref, segq_ref, cnt_ref):
    i = pl.program_id(0)

    @pl.when(i == 0)
    def _():
        segs_ref[...] = jnp.zeros_like(segs_ref)
        segq_ref[...] = jnp.zeros_like(segq_ref)
        cnt_ref[...] = jnp.zeros_like(cnt_ref)

    deg = deg_ref[0, :, 0] + deg_ref[1, :, 0] + 1.0
    dinv = lax.rsqrt(deg)[:, None]
    out = (acc_ref[0] + acc_ref[1]) * dinv + b_ref[...]
    out_ref[...] = out
    oh = (batch_ref[...] == lax.broadcasted_iota(jnp.int32, (1, G), 1)
          ).astype(jnp.float32)
    segs_ref[...] += jnp.dot(oh.T, out, preferred_element_type=jnp.float32)
    segq_ref[...] += jnp.dot(oh.T, out * out,
                             preferred_element_type=jnp.float32)
    cnt_ref[...] += jnp.sum(oh, axis=0)[:, None]


def _tc_stats(acc, deg2, b2, batch2):
    BN = 2048
    return pl.pallas_call(
        _stats_body,
        grid=(NP // BN,),
        in_specs=[
            pl.BlockSpec((NC, BN, D), lambda i: (0, i, 0)),
            pl.BlockSpec((NC, BN, DEGW), lambda i: (0, i, 0)),
            pl.BlockSpec((1, D), lambda i: (0, 0)),
            pl.BlockSpec((BN, 1), lambda i: (i, 0)),
        ],
        out_specs=[
            pl.BlockSpec((BN, D), lambda i: (i, 0)),
            pl.BlockSpec((G, D), lambda i: (0, 0)),
            pl.BlockSpec((G, D), lambda i: (0, 0)),
            pl.BlockSpec((G, 1), lambda i: (0, 0)),
        ],
        out_shape=[
            jax.ShapeDtypeStruct((NP, D), jnp.float32),
            jax.ShapeDtypeStruct((G, D), jnp.float32),
            jax.ShapeDtypeStruct((G, D), jnp.float32),
            jax.ShapeDtypeStruct((G, 1), jnp.float32),
        ],
    )(acc, deg2, b2, batch2)


def _apply_body(out_ref, segs_ref, segq_ref, cnt_ref, batch_ref,
                gw_ref, gb_ref, gs_ref, y_ref):
    cnt = jnp.maximum(cnt_ref[...], 1.0)
    mean = segs_ref[...] / cnt
    scale = gs_ref[...]
    m2 = mean * mean
    var = segq_ref[...] / cnt - m2 * scale * (2.0 - scale)
    rstd = lax.rsqrt(var + 1e-5)
    ms = mean * scale
    oh = (batch_ref[...] == lax.broadcasted_iota(jnp.int32, (1, G), 1)
          ).astype(jnp.float32)
    centered = out_ref[...] - jnp.dot(oh, ms, preferred_element_type=jnp.float32)
    y = gw_ref[...] * centered * jnp.dot(oh, rstd,
                                         preferred_element_type=jnp.float32)
    y_ref[...] = jnp.maximum(y + gb_ref[...], 0.0)


def _tc_apply(out, segs, segq, cnt, batch2, gw2, gb2, gs2):
    BN = 2048
    return pl.pallas_call(
        _apply_body,
        grid=(NP // BN,),
        in_specs=[
            pl.BlockSpec((BN, D), lambda i: (i, 0)),
            pl.BlockSpec((G, D), lambda i: (0, 0)),
            pl.BlockSpec((G, D), lambda i: (0, 0)),
            pl.BlockSpec((G, 1), lambda i: (0, 0)),
            pl.BlockSpec((BN, 1), lambda i: (i, 0)),
            pl.BlockSpec((1, D), lambda i: (0, 0)),
            pl.BlockSpec((1, D), lambda i: (0, 0)),
            pl.BlockSpec((1, D), lambda i: (0, 0)),
        ],
        out_specs=pl.BlockSpec((BN, D), lambda i: (i, 0)),
        out_shape=jax.ShapeDtypeStruct((NP, D), jnp.float32),
    )(out, segs, segq, cnt, batch2, gw2, gb2, gs2)


def kernel(x, W, b, gn_weight, gn_bias, gn_mean_scale, edge_index, batch):
    src = edge_index[0]
    dst = edge_index[1]
    xp = jnp.pad(x, ((0, NP - N), (0, 0)))
    ones_kw = jnp.ones((K, DEGW), jnp.float32)
    zeros_nw = jnp.zeros((NP, DEGW), jnp.float32)
    deg2 = _sc_degree(dst, ones_kw, zeros_nw)
    hprime = _tc_matmul(xp, W, deg2)
    zeros_nd = jnp.zeros((NP, D), jnp.float32)
    acc = _sc_scatter(hprime, src, dst, zeros_nd)
    batch2 = jnp.pad(batch, (0, NP - N), constant_values=G)[:, None]
    out, segs, segq, cnt = _tc_stats(acc, deg2, b[None, :], batch2)
    y = _tc_apply(out, segs, segq, cnt, batch2, gn_weight[None, :],
                  gn_bias[None, :], gn_mean_scale[None, :])
    return y[:N]
